# Initial kernel scaffold; baseline (speedup 1.0000x reference)
import jax
import jax.numpy as jnp
from jax.experimental import pallas as pl


def kernel(x_author, x_paper, ei_writes, ei_written_by, W_nbr_w_0, W_root_w_0, b_w_0, W_nbr_wb_0, W_root_wb_0, b_wb_0, W_nbr_w_1, W_root_w_1, b_w_1, W_nbr_wb_1, W_root_wb_1, b_wb_1, W_nbr_w_2, W_root_w_2, b_w_2, W_nbr_wb_2, W_root_wb_2, b_wb_2, W_lin, b_lin):
    raise NotImplementedError("write your pallas kernel here")


# speedup vs baseline: 4.0462x; 4.0462x over previous
import functools

import jax
import jax.numpy as jnp
from jax import lax
from jax.experimental import pallas as pl
from jax.experimental.pallas import tpu as pltpu
from jax.experimental.pallas import tpu_sc as plsc

N = 10000
E = 320000
HID = 256
R = 10240
NS = 16
ET = E // NS
NCHUNK = ET // 128
REM = ET - NCHUNK * 128
RPT = R // NS


def _sc_mesh():
    return plsc.VectorSubcoreMesh(core_axis_name="c", subcore_axis_name="s")


@functools.cache
def _make_segsum_es(W):
    ET2 = E // 2 // NS
    NC2 = ET2 // 128
    REM2 = ET2 - NC2 * 128
    out_t = (jax.ShapeDtypeStruct((R, W), jnp.float32),
             jax.ShapeDtypeStruct((R, W), jnp.float32))

    @functools.partial(
        pl.kernel, out_type=out_t, mesh=_sc_mesh(),
        compiler_params=pltpu.CompilerParams(needs_layout_passes=False),
        scratch_types=(
            pltpu.VMEM_SHARED((R, W), jnp.float32),
            pltpu.VMEM((16, W), jnp.float32),
            pltpu.VMEM((128,), jnp.int32),
            pltpu.VMEM((128,), jnp.int32),
            pltpu.VMEM((128, W), jnp.float32),
            pltpu.VMEM((REM2,), jnp.int32),
            pltpu.VMEM((REM2,), jnp.int32),
            pltpu.VMEM((REM2, W), jnp.float32),
            pltpu.SemaphoreType.DMA,
        ),
    )
    def segsum(src, dst, tab, out0, out1,
               acc, zbuf, sidx, didx, rows, sidx_r, didx_r, rows_r, sem):
        cid = lax.axis_index("c")
        sid = lax.axis_index("s")
        t0 = cid * (E // 2) + sid * ET2

        z16 = jnp.zeros((16,), jnp.float32)
        for r in range(16):
            for j in range(W // 16):
                zbuf[r, pl.ds(j * 16, 16)] = z16

        def zero_body(i, _):
            pltpu.sync_copy(zbuf, acc.at[pl.ds(sid * RPT + i * 16, 16)])
            return 0
        lax.fori_loop(0, RPT // 16, zero_body, 0)
        plsc.subcore_barrier()

        def run(out):
            def body(c, _):
                base = t0 + c * 128
                pltpu.sync_copy(src.at[pl.ds(base, 128)], sidx)
                pltpu.sync_copy(dst.at[pl.ds(base, 128)], didx)
                pltpu.async_copy(tab.at[sidx], rows, sem).wait()
                pltpu.sync_copy(rows, acc.at[didx], add=True)
                return 0
            lax.fori_loop(0, NC2, body, 0)
            base = t0 + NC2 * 128
            pltpu.sync_copy(src.at[pl.ds(base, REM2)], sidx_r)
            pltpu.sync_copy(dst.at[pl.ds(base, REM2)], didx_r)
            pltpu.async_copy(tab.at[sidx_r], rows_r, sem).wait()
            pltpu.sync_copy(rows_r, acc.at[didx_r], add=True)
            plsc.subcore_barrier()
            pltpu.sync_copy(acc.at[pl.ds(sid * RPT, RPT)],
                            out.at[pl.ds(sid * RPT, RPT)])

        @pl.when(cid == 0)
        def _():
            run(out0)

        @pl.when(cid == 1)
        def _():
            run(out1)

    return segsum


@functools.cache
def _make_segsum(W):
    out_t = (jax.ShapeDtypeStruct((R, W), jnp.float32),
             jax.ShapeDtypeStruct((R, W), jnp.float32))

    @functools.partial(
        pl.kernel, out_type=out_t, mesh=_sc_mesh(),
        compiler_params=pltpu.CompilerParams(needs_layout_passes=False),
        scratch_types=(
            pltpu.VMEM_SHARED((R, W), jnp.float32),
            pltpu.VMEM((16, W), jnp.float32),
            pltpu.VMEM((128,), jnp.int32),
            pltpu.VMEM((128,), jnp.int32),
            pltpu.VMEM((128, W), jnp.float32),
            pltpu.VMEM((REM,), jnp.int32),
            pltpu.VMEM((REM,), jnp.int32),
            pltpu.VMEM((REM, W), jnp.float32),
            pltpu.SemaphoreType.DMA,
        ),
    )
    def segsum(src, dst, tab_lo, tab_hi, out_lo, out_hi,
               acc, zbuf, sidx, didx, rows, sidx_r, didx_r, rows_r, sem):
        cid = lax.axis_index("c")
        sid = lax.axis_index("s")
        t0 = sid * ET

        z16 = jnp.zeros((16,), jnp.float32)
        for r in range(16):
            for j in range(W // 16):
                zbuf[r, pl.ds(j * 16, 16)] = z16

        def zero_body(i, _):
            pltpu.sync_copy(zbuf, acc.at[pl.ds(sid * RPT + i * 16, 16)])
            return 0
        lax.fori_loop(0, RPT // 16, zero_body, 0)
        plsc.subcore_barrier()

        def run(tab, out):
            def body(c, _):
                base = t0 + c * 128
                pltpu.sync_copy(src.at[pl.ds(base, 128)], sidx)
                pltpu.sync_copy(dst.at[pl.ds(base, 128)], didx)
                pltpu.async_copy(tab.at[sidx], rows, sem).wait()
                pltpu.sync_copy(rows, acc.at[didx], add=True)
                return 0
            lax.fori_loop(0, NCHUNK, body, 0)
            base = t0 + NCHUNK * 128
            pltpu.sync_copy(src.at[pl.ds(base, REM)], sidx_r)
            pltpu.sync_copy(dst.at[pl.ds(base, REM)], didx_r)
            pltpu.async_copy(tab.at[sidx_r], rows_r, sem).wait()
            pltpu.sync_copy(rows_r, acc.at[didx_r], add=True)
            plsc.subcore_barrier()
            pltpu.sync_copy(acc.at[pl.ds(sid * RPT, RPT)],
                            out.at[pl.ds(sid * RPT, RPT)])

        @pl.when(cid == 0)
        def _():
            run(tab_lo, out_lo)

        @pl.when(cid == 1)
        def _():
            run(tab_hi, out_hi)

    return segsum


@functools.cache
def _make_invdeg():
    out_t = jax.ShapeDtypeStruct((R,), jnp.float32)

    @functools.partial(
        pl.kernel, out_type=out_t, mesh=_sc_mesh(),
        compiler_params=pltpu.CompilerParams(needs_layout_passes=False),
        scratch_types=(
            pltpu.VMEM((R,), jnp.float32),
            pltpu.VMEM_SHARED((NS, R), jnp.float32),
            pltpu.VMEM((128,), jnp.int32),
            pltpu.VMEM((REM,), jnp.int32),
            pltpu.VMEM((RPT,), jnp.float32),
            pltpu.VMEM((RPT,), jnp.float32),
        ),
    )
    def invdeg(dst, out, dvm, part, didx, didx_r, pbuf, tot):
        cid = lax.axis_index("c")
        sid = lax.axis_index("s")
        t0 = sid * ET
        ones16 = jnp.ones((16,), jnp.float32)
        z16 = jnp.zeros((16,), jnp.float32)

        def zero_body(i, _):
            dvm[pl.ds(i * 16, 16)] = z16
            return 0
        lax.fori_loop(0, R // 16, zero_body, 0)

        def body(c, _):
            base = t0 + c * 128
            pltpu.sync_copy(dst.at[pl.ds(base, 128)], didx)
            for j in range(8):
                iv = didx[pl.ds(j * 16, 16)]
                plsc.addupdate_scatter(dvm, [iv], ones16)
            return 0
        lax.fori_loop(0, NCHUNK, body, 0)
        base = t0 + NCHUNK * 128
        pltpu.sync_copy(dst.at[pl.ds(base, REM)], didx_r)
        for j in range(REM // 16):
            iv = didx_r[pl.ds(j * 16, 16)]
            plsc.addupdate_scatter(dvm, [iv], ones16)

        pltpu.sync_copy(dvm, part.at[sid])
        plsc.subcore_barrier()

        r0 = sid * RPT
        def acc_body(p, _):
            pltpu.sync_copy(part.at[p, pl.ds(r0, RPT)], pbuf)
            for j in range(RPT // 16):
                tot[pl.ds(j * 16, 16)] = (tot[pl.ds(j * 16, 16)]
                                          + pbuf[pl.ds(j * 16, 16)])
            return 0
        for j in range(RPT // 16):
            tot[pl.ds(j * 16, 16)] = z16
        lax.fori_loop(0, NS, acc_body, 0)

        for j in range(RPT // 16):
            v = tot[pl.ds(j * 16, 16)]
            tot[pl.ds(j * 16, 16)] = 1.0 / jnp.maximum(v, 1.0)

        @pl.when(cid == 0)
        def _():
            pltpu.sync_copy(tot, out.at[pl.ds(r0, RPT)])

    return invdeg


_BLK = 400


def _elu(z):
    return jnp.where(z > 0, z, jnp.exp(jnp.minimum(z, 0.0)) - 1.0)


def _tc_layer(slo, shi, inv, xlo, xhi, Wn, Wr, b):
    W = slo.shape[1]
    din = 2 * W
    h = HID // 2

    def body(slo_r, shi_r, inv_r, xlo_r, xhi_r, wn_r, wr_r, b_r, olo_r, ohi_r):
        s = jnp.concatenate([slo_r[...], shi_r[...]], axis=1) * inv_r[...]
        x = jnp.concatenate([xlo_r[...], xhi_r[...]], axis=1)
        z = (jnp.dot(s, wn_r[...], preferred_element_type=jnp.float32)
             + jnp.dot(x, wr_r[...], preferred_element_type=jnp.float32)
             + b_r[...])
        y = _elu(z)
        olo_r[...] = y[:, :h]
        ohi_r[...] = y[:, h:]

    grid = (N // _BLK,)
    return pl.pallas_call(
        body,
        grid=grid,
        in_specs=[
            pl.BlockSpec((_BLK, W), lambda i: (i, 0)),
            pl.BlockSpec((_BLK, W), lambda i: (i, 0)),
            pl.BlockSpec((_BLK, 1), lambda i: (i, 0)),
            pl.BlockSpec((_BLK, W), lambda i: (i, 0)),
            pl.BlockSpec((_BLK, W), lambda i: (i, 0)),
            pl.BlockSpec((din, HID), lambda i: (0, 0)),
            pl.BlockSpec((din, HID), lambda i: (0, 0)),
            pl.BlockSpec((1, HID), lambda i: (0, 0)),
        ],
        out_specs=[
            pl.BlockSpec((_BLK, h), lambda i: (i, 0)),
            pl.BlockSpec((_BLK, h), lambda i: (i, 0)),
        ],
        out_shape=[
            jax.ShapeDtypeStruct((N, h), jnp.float32),
            jax.ShapeDtypeStruct((N, h), jnp.float32),
        ],
    )(slo, shi, inv, xlo, xhi, Wn, Wr, b.reshape(1, HID))


def _tc_layer0(p0, p1, inv, x, Wn, Wr, b):
    din = x.shape[1]
    h = HID // 2

    def body(p0_r, p1_r, inv_r, x_r, wn_r, wr_r, b_r, olo_r, ohi_r):
        s = (p0_r[...] + p1_r[...]) * inv_r[...]
        z = (jnp.dot(s, wn_r[...], preferred_element_type=jnp.float32)
             + jnp.dot(x_r[...], wr_r[...], preferred_element_type=jnp.float32)
             + b_r[...])
        y = _elu(z)
        olo_r[...] = y[:, :h]
        ohi_r[...] = y[:, h:]

    grid = (N // _BLK,)
    return pl.pallas_call(
        body,
        grid=grid,
        in_specs=[
            pl.BlockSpec((_BLK, din), lambda i: (i, 0)),
            pl.BlockSpec((_BLK, din), lambda i: (i, 0)),
            pl.BlockSpec((_BLK, 1), lambda i: (i, 0)),
            pl.BlockSpec((_BLK, din), lambda i: (i, 0)),
            pl.BlockSpec((din, HID), lambda i: (0, 0)),
            pl.BlockSpec((din, HID), lambda i: (0, 0)),
            pl.BlockSpec((1, HID), lambda i: (0, 0)),
        ],
        out_specs=[
            pl.BlockSpec((_BLK, h), lambda i: (i, 0)),
            pl.BlockSpec((_BLK, h), lambda i: (i, 0)),
        ],
        out_shape=[
            jax.ShapeDtypeStruct((N, h), jnp.float32),
            jax.ShapeDtypeStruct((N, h), jnp.float32),
        ],
    )(p0, p1, inv, x, Wn, Wr, b.reshape(1, HID))


def _tc_final(slo, shi, inv, xlo, xhi, Wn, Wr, b, Wl, bl):
    W = slo.shape[1]
    din = 2 * W
    nout = Wl.shape[1]

    def body(slo_r, shi_r, inv_r, xlo_r, xhi_r, wn_r, wr_r, b_r, wl_r, bl_r,
             o_r):
        s = jnp.concatenate([slo_r[...], shi_r[...]], axis=1) * inv_r[...]
        x = jnp.concatenate([xlo_r[...], xhi_r[...]], axis=1)
        z = (jnp.dot(s, wn_r[...], preferred_element_type=jnp.float32)
             + jnp.dot(x, wr_r[...], preferred_element_type=jnp.float32)
             + b_r[...])
        y = _elu(z)
        o_r[...] = (jnp.dot(y, wl_r[...], preferred_element_type=jnp.float32)
                    + bl_r[...])

    grid = (N // _BLK,)
    return pl.pallas_call(
        body,
        grid=grid,
        in_specs=[
            pl.BlockSpec((_BLK, W), lambda i: (i, 0)),
            pl.BlockSpec((_BLK, W), lambda i: (i, 0)),
            pl.BlockSpec((_BLK, 1), lambda i: (i, 0)),
            pl.BlockSpec((_BLK, W), lambda i: (i, 0)),
            pl.BlockSpec((_BLK, W), lambda i: (i, 0)),
            pl.BlockSpec((din, HID), lambda i: (0, 0)),
            pl.BlockSpec((din, HID), lambda i: (0, 0)),
            pl.BlockSpec((1, HID), lambda i: (0, 0)),
            pl.BlockSpec((HID, nout), lambda i: (0, 0)),
            pl.BlockSpec((1, nout), lambda i: (0, 0)),
        ],
        out_specs=pl.BlockSpec((_BLK, nout), lambda i: (i, 0)),
        out_shape=jax.ShapeDtypeStruct((N, nout), jnp.float32),
    )(slo, shi, inv, xlo, xhi, Wn, Wr, b.reshape(1, HID), Wl,
      bl.reshape(1, nout))


def kernel(x_author, x_paper, ei_writes, ei_written_by,
           W_nbr_w_0, W_root_w_0, b_w_0, W_nbr_wb_0, W_root_wb_0, b_wb_0,
           W_nbr_w_1, W_root_w_1, b_w_1, W_nbr_wb_1, W_root_wb_1, b_wb_1,
           W_nbr_w_2, W_root_w_2, b_w_2, W_nbr_wb_2, W_root_wb_2, b_wb_2,
           W_lin, b_lin):
    src_w, dst_w = ei_writes[0], ei_writes[1]
    src_wb, dst_wb = ei_written_by[0], ei_written_by[1]

    invdeg = _make_invdeg()
    inv_w = invdeg(dst_w).reshape(R, 1)[:N]
    inv_wb = invdeg(dst_wb).reshape(R, 1)[:N]

    ss0 = _make_segsum_es(x_author.shape[1])
    sp = ss0(src_w, dst_w, x_author)
    sa = ss0(src_wb, dst_wb, x_paper)
    xp1 = _tc_layer0(*sp, inv_w, x_paper, W_nbr_w_0, W_root_w_0, b_w_0)
    xa1 = _tc_layer0(*sa, inv_wb, x_author, W_nbr_wb_0, W_root_wb_0, b_wb_0)

    ss = _make_segsum(HID // 2)
    sp = ss(src_w, dst_w, *xa1)
    sa = ss(src_wb, dst_wb, *xp1)
    xp2 = _tc_layer(*sp, inv_w, *xp1, W_nbr_w_1, W_root_w_1, b_w_1)
    xa2 = _tc_layer(*sa, inv_wb, *xa1, W_nbr_wb_1, W_root_wb_1, b_wb_1)

    sa = ss(src_wb, dst_wb, *xp2)
    return _tc_final(*sa, inv_wb, *xa2, W_nbr_wb_2, W_root_wb_2, b_wb_2,
                     W_lin, b_lin)
